# baseline (speedup 1.0000x reference)
import functools

import jax
import jax.numpy as jnp
from jax import lax
from jax.experimental import pallas as pl
from jax.experimental.pallas import tpu as pltpu
from jax.experimental.pallas import tpu_sc as plsc

NC = 2
NS = 16
CHUNK = 64

@functools.cache
def _mesh():
    return plsc.VectorSubcoreMesh(
        core_axis_name="c", subcore_axis_name="s", num_cores=NC, num_subcores=NS
    )


def _prop_body(cpt, np_, f, hs_hbm, src_hbm, dst_hbm, zeros_hbm, out_hbm,
               srcv, dstv, rows0, rows1, acc, sem0, sem1):
    c = lax.axis_index("c")
    s = lax.axis_index("s")
    wid = c * NS + s
    rows = np_ // NS
    pltpu.sync_copy(zeros_hbm.at[pl.ds(s * rows, rows)], acc.at[pl.ds(s * rows, rows)])
    pltpu.sync_copy(src_hbm.at[pl.ds(wid * cpt, cpt)], srcv)
    pltpu.sync_copy(dst_hbm.at[pl.ds(wid * cpt, cpt)], dstv)
    plsc.subcore_barrier()

    pltpu.async_copy(hs_hbm.at[srcv.at[0]], rows0, sem0)

    def body(i, carry):
        j = 2 * i
        pltpu.make_async_copy(hs_hbm.at[srcv.at[j]], rows0, sem0).wait()
        pltpu.async_copy(hs_hbm.at[srcv.at[j + 1]], rows1, sem1)
        pltpu.sync_copy(rows0, acc.at[dstv.at[j]], add=True)
        pltpu.make_async_copy(hs_hbm.at[srcv.at[j + 1]], rows1, sem1).wait()

        @pl.when(j + 2 < cpt)
        def _():
            pltpu.async_copy(hs_hbm.at[srcv.at[j + 2]], rows0, sem0)

        pltpu.sync_copy(rows1, acc.at[dstv.at[j + 1]], add=True)
        return carry

    lax.fori_loop(0, cpt // 2, body, 0)
    plsc.subcore_barrier()
    pltpu.sync_copy(acc.at[pl.ds(s * rows, rows)], out_hbm.at[c, pl.ds(s * rows, rows)])


def _sc_propagate(hs, src2d, dst2d, zeros_f, np_):
    f = hs.shape[1]
    cpt = src2d.shape[0] // (NC * NS)
    k = pl.kernel(
        functools.partial(_prop_body, cpt, np_, f),
        out_type=jax.ShapeDtypeStruct((NC, np_, f), jnp.float32),
        mesh=_mesh(),
        scratch_types=[
            pltpu.VMEM((cpt, CHUNK), jnp.int32),
            pltpu.VMEM((cpt, CHUNK), jnp.int32),
            pltpu.VMEM((CHUNK, f), jnp.float32),
            pltpu.VMEM((CHUNK, f), jnp.float32),
            pltpu.VMEM_SHARED((np_, f), jnp.float32),
            pltpu.SemaphoreType.DMA,
            pltpu.SemaphoreType.DMA,
        ],
        compiler_params=pltpu.CompilerParams(use_tc_tiling_on_sc=False),
    )
    return k(hs, src2d, dst2d, zeros_f)


_BLK = 1024


def _prep_body(g, deg2_ref, batch_ref, lmi_ref, x_ref, w0_ref,
               dinv_ref, c_ref, diag_ref, hs_ref, acc_ref):
    deg = deg2_ref[0] + deg2_ref[1]
    dinv = jnp.where(deg > 0.0, lax.rsqrt(jnp.where(deg > 0.0, deg, 1.0)), 0.0)
    b = batch_ref[...]
    onehot = (b == lax.broadcasted_iota(jnp.int32, (b.shape[0], g), 1)).astype(
        jnp.float32
    )
    lam2 = onehot @ lmi_ref[...]
    dinv_ref[...] = dinv
    diag_ref[...] = lam2 - 1.0
    c_ref[...] = lam2 * dinv
    x = x_ref[...]
    hs_ref[...] = (lam2 * dinv) * x
    acc_ref[...] = jnp.dot(x, w0_ref[...], preferred_element_type=jnp.float32)


def _tc_prep(deg2, batch_pad, lmi, x, w0, np_):
    g = lmi.shape[0]
    d = x.shape[1]
    fo = w0.shape[1]
    nb = np_ // _BLK
    return pl.pallas_call(
        functools.partial(_prep_body, g),
        grid=(nb,),
        in_specs=[
            pl.BlockSpec((NC, _BLK, 1), lambda i: (0, i, 0)),
            pl.BlockSpec((_BLK, 1), lambda i: (i, 0)),
            pl.BlockSpec((g, 1), lambda i: (0, 0)),
            pl.BlockSpec((_BLK, d), lambda i: (i, 0)),
            pl.BlockSpec((d, fo), lambda i: (0, 0)),
        ],
        out_specs=[
            pl.BlockSpec((_BLK, 1), lambda i: (i, 0)),
            pl.BlockSpec((_BLK, 1), lambda i: (i, 0)),
            pl.BlockSpec((_BLK, 1), lambda i: (i, 0)),
            pl.BlockSpec((_BLK, d), lambda i: (i, 0)),
            pl.BlockSpec((_BLK, fo), lambda i: (i, 0)),
        ],
        out_shape=[
            jax.ShapeDtypeStruct((np_, 1), jnp.float32),
            jax.ShapeDtypeStruct((np_, 1), jnp.float32),
            jax.ShapeDtypeStruct((np_, 1), jnp.float32),
            jax.ShapeDtypeStruct((np_, d), jnp.float32),
            jax.ShapeDtypeStruct((np_, fo), jnp.float32),
        ],
    )(deg2, batch_pad, lmi, x, w0)


def _step_body(alpha, beta, emit_hs, parts_ref, tp_ref, tp2_ref, dinv_ref,
               c_ref, diag_ref, wk_ref, accin_ref, tx_ref, hs_ref, accout_ref):
    agg = parts_ref[0] + parts_ref[1]
    tp = tp_ref[...]
    lh = diag_ref[...] * tp - dinv_ref[...] * agg
    tx = alpha * lh - beta * tp2_ref[...] if beta else alpha * lh
    tx_ref[...] = tx
    if emit_hs:
        hs_ref[...] = c_ref[...] * tx
    else:
        hs_ref[...] = tx
    accout_ref[...] = accin_ref[...] + jnp.dot(
        tx, wk_ref[...], preferred_element_type=jnp.float32
    )


def _tc_step(parts, tp, tp2, dinv, cvec, diag, wk, accin, alpha, beta,
             emit_hs, np_):
    f = tp.shape[1]
    fo = wk.shape[1]
    nb = np_ // _BLK
    return pl.pallas_call(
        functools.partial(_step_body, alpha, beta, emit_hs),
        grid=(nb,),
        in_specs=[
            pl.BlockSpec((NC, _BLK, f), lambda i: (0, i, 0)),
            pl.BlockSpec((_BLK, f), lambda i: (i, 0)),
            pl.BlockSpec((_BLK, f), lambda i: (i, 0)),
            pl.BlockSpec((_BLK, 1), lambda i: (i, 0)),
            pl.BlockSpec((_BLK, 1), lambda i: (i, 0)),
            pl.BlockSpec((_BLK, 1), lambda i: (i, 0)),
            pl.BlockSpec((f, fo), lambda i: (0, 0)),
            pl.BlockSpec((_BLK, fo), lambda i: (i, 0)),
        ],
        out_specs=[
            pl.BlockSpec((_BLK, f), lambda i: (i, 0)),
            pl.BlockSpec((_BLK, f), lambda i: (i, 0)),
            pl.BlockSpec((_BLK, fo), lambda i: (i, 0)),
        ],
        out_shape=[
            jax.ShapeDtypeStruct((np_, f), jnp.float32),
            jax.ShapeDtypeStruct((np_, f), jnp.float32),
            jax.ShapeDtypeStruct((np_, fo), jnp.float32),
        ],
    )(parts, tp, tp2, dinv, cvec, diag, wk, accin)


def _fin_body(has_next, parts_ref, tp_ref, tp2_ref, dinv_ref, c_ref, diag_ref,
              wk_ref, accin_ref, bias_ref, wn_ref, h_ref, hs_ref, accn_ref):
    agg = parts_ref[0] + parts_ref[1]
    lh = diag_ref[...] * tp_ref[...] - dinv_ref[...] * agg
    tx = 2.0 * lh - tp2_ref[...]
    o = accin_ref[...] + jnp.dot(tx, wk_ref[...], preferred_element_type=jnp.float32)
    h = jnp.maximum(o + bias_ref[...], 0.0)
    h_ref[...] = h
    if has_next:
        hs_ref[...] = c_ref[...] * h
        accn_ref[...] = jnp.dot(h, wn_ref[...], preferred_element_type=jnp.float32)


def _tc_finish(parts, tp, tp2, dinv, cvec, diag, wk, accin, bias, wnext, np_):
    f = tp.shape[1]
    fo = wk.shape[1]
    fn = wnext.shape[1]
    nb = np_ // _BLK
    return pl.pallas_call(
        functools.partial(_fin_body, True),
        grid=(nb,),
        in_specs=[
            pl.BlockSpec((NC, _BLK, f), lambda i: (0, i, 0)),
            pl.BlockSpec((_BLK, f), lambda i: (i, 0)),
            pl.BlockSpec((_BLK, f), lambda i: (i, 0)),
            pl.BlockSpec((_BLK, 1), lambda i: (i, 0)),
            pl.BlockSpec((_BLK, 1), lambda i: (i, 0)),
            pl.BlockSpec((_BLK, 1), lambda i: (i, 0)),
            pl.BlockSpec((f, fo), lambda i: (0, 0)),
            pl.BlockSpec((_BLK, fo), lambda i: (i, 0)),
            pl.BlockSpec((1, fo), lambda i: (0, 0)),
            pl.BlockSpec((fo, fn), lambda i: (0, 0)),
        ],
        out_specs=[
            pl.BlockSpec((_BLK, fo), lambda i: (i, 0)),
            pl.BlockSpec((_BLK, fo), lambda i: (i, 0)),
            pl.BlockSpec((_BLK, fn), lambda i: (i, 0)),
        ],
        out_shape=[
            jax.ShapeDtypeStruct((np_, fo), jnp.float32),
            jax.ShapeDtypeStruct((np_, fo), jnp.float32),
            jax.ShapeDtypeStruct((np_, fn), jnp.float32),
        ],
    )(parts, tp, tp2, dinv, cvec, diag, wk, accin, bias, wnext)


def _pool_body(g, nb, h_ref, batch_ref, a1w_ref, a1b_ref, a2w_ref, a2b_ref,
               out_ref, acc_ref):
    i = pl.program_id(0)

    @pl.when(i == 0)
    def _init():
        acc_ref[...] = jnp.full_like(acc_ref, -jnp.inf)

    h = h_ref[...]
    b = batch_ref[...]
    for gg in range(g):
        sel = jnp.where(b == gg, h, -jnp.inf)
        acc_ref[gg, :] = jnp.maximum(acc_ref[gg, :], jnp.max(sel, axis=0))

    @pl.when(i == nb - 1)
    def _fin():
        gmax = acc_ref[...]
        gmax = jnp.where(jnp.isfinite(gmax), gmax, 0.0)
        z = jnp.maximum(
            jnp.dot(gmax, a1w_ref[...], preferred_element_type=jnp.float32)
            + a1b_ref[...],
            0.0,
        )
        out_ref[...] = (
            jnp.dot(z, a2w_ref[...], preferred_element_type=jnp.float32)
            + a2b_ref[...]
        )


def _tc_pool(h3, batch_pad, a1w, a1b, a2w, a2b, g, np_):
    f = h3.shape[1]
    nb = np_ // _BLK
    return pl.pallas_call(
        functools.partial(_pool_body, g, nb),
        grid=(nb,),
        in_specs=[
            pl.BlockSpec((_BLK, f), lambda i: (i, 0)),
            pl.BlockSpec((_BLK, 1), lambda i: (i, 0)),
            pl.BlockSpec((f, 16), lambda i: (0, 0)),
            pl.BlockSpec((1, 16), lambda i: (0, 0)),
            pl.BlockSpec((16, 1), lambda i: (0, 0)),
            pl.BlockSpec((1, 1), lambda i: (0, 0)),
        ],
        out_specs=pl.BlockSpec((g, 1), lambda i: (0, 0)),
        out_shape=jax.ShapeDtypeStruct((g, 1), jnp.float32),
        scratch_shapes=[pltpu.VMEM((g, f), jnp.float32)],
    )(h3, batch_pad, a1w, a1b, a2w, a2b)


def kernel(x, edge_index, batch, lmax, W1, b1, W2, b2, W3, b3, A1w, A1b, A2w, A2b):
    n, d = x.shape
    e = edge_index.shape[1]
    g = lmax.shape[0]
    s_order = W1.shape[0]

    np_ = ((n + 16 + _BLK - 1) // _BLK) * _BLK
    unit = CHUNK * NC * NS * 8
    ecp = ((e + unit - 1) // unit) * NC * NS * 8
    pad_e = ecp * CHUNK - e

    src = edge_index[0]
    dst = edge_index[1]
    pad_idx = n + (jnp.arange(pad_e, dtype=jnp.int32) % 16)
    src2d = jnp.concatenate([src, pad_idx]).reshape(ecp, CHUNK)
    dst2d = jnp.concatenate([dst, pad_idx]).reshape(ecp, CHUNK)

    x_pad = jnp.zeros((np_, d), jnp.float32).at[:n].set(x)
    batch_pad = jnp.full((np_, 1), g, jnp.int32).at[:n, 0].set(batch)
    lmi = (2.0 / lmax).reshape(g, 1)

    zeros = {
        f: jnp.zeros((np_, f), jnp.float32)
        for f in {16, W1.shape[1], W2.shape[1], W3.shape[1]}
    }

    ones16 = jnp.ones((np_, 16), jnp.float32)
    deg16 = _sc_propagate(ones16, src2d, src2d, zeros[16], np_)
    deg2 = deg16[:, :, :1]
    dinv, cvec, diag, hs, acc = _tc_prep(deg2, batch_pad, lmi, x_pad, W1[0], np_)

    layers = [
        (W1, b1, W2[0]),
        (W2, b2, W3[0]),
        (W3, b3, None),
    ]
    h = x_pad
    for li, (W, b, wnext) in enumerate(layers):
        f_in = W.shape[1]
        tp2 = h
        tp = None
        for k in range(1, s_order):
            parts = _sc_propagate(hs, src2d, dst2d, zeros[f_in], np_)
            if k == 1:
                tp, hs, acc = _tc_step(
                    parts, tp2, tp2, dinv, cvec, diag, W[1], acc,
                    1.0, 0.0, True, np_,
                )
            elif k < s_order - 1:
                tx, hs, acc = _tc_step(
                    parts, tp, tp2, dinv, cvec, diag, W[k], acc,
                    2.0, 1.0, True, np_,
                )
                tp2, tp = tp, tx
            else:
                bias = b.reshape(1, -1)
                wn = wnext if wnext is not None else jnp.zeros(
                    (W.shape[2], 8), jnp.float32
                )
                h, hs, acc = _tc_finish(
                    parts, tp, tp2, dinv, cvec, diag, W[k], acc, bias, wn, np_,
                )

    return _tc_pool(h, batch_pad, A1w, A1b.reshape(1, -1), A2w,
                    A2b.reshape(1, -1), g, np_)


# speedup vs baseline: 1.0961x; 1.0961x over previous
import functools

import jax
import jax.numpy as jnp
from jax import lax
from jax.experimental import pallas as pl
from jax.experimental.pallas import tpu as pltpu
from jax.experimental.pallas import tpu_sc as plsc

NC = 2
NS = 16
CHUNK = 128


@functools.cache
def _mesh():
    return plsc.VectorSubcoreMesh(
        core_axis_name="c", subcore_axis_name="s", num_cores=NC, num_subcores=NS
    )


def _prop_body(cpt, np_, f, hs_hbm, src_hbm, dst_hbm, zeros_hbm, out_hbm,
               srcv, dstv, rowsv, acc, sem):
    c = lax.axis_index("c")
    s = lax.axis_index("s")
    wid = c * NS + s
    rows = np_ // NS
    pltpu.sync_copy(zeros_hbm.at[pl.ds(s * rows, rows)], acc.at[pl.ds(s * rows, rows)])
    pltpu.sync_copy(src_hbm.at[pl.ds(wid * cpt, cpt)], srcv)
    pltpu.sync_copy(dst_hbm.at[pl.ds(wid * cpt, cpt)], dstv)
    plsc.subcore_barrier()

    def body(j, carry):
        pltpu.async_copy(hs_hbm.at[srcv.at[j]], rowsv, sem).wait()
        pltpu.sync_copy(rowsv, acc.at[dstv.at[j]], add=True)
        return carry

    lax.fori_loop(0, cpt, body, 0)
    plsc.subcore_barrier()
    pltpu.sync_copy(acc.at[pl.ds(s * rows, rows)], out_hbm.at[c, pl.ds(s * rows, rows)])


def _sc_propagate(hs, src2d, dst2d, zeros_f, np_):
    f = hs.shape[1]
    cpt = src2d.shape[0] // (NC * NS)
    k = pl.kernel(
        functools.partial(_prop_body, cpt, np_, f),
        out_type=jax.ShapeDtypeStruct((NC, np_, f), jnp.float32),
        mesh=_mesh(),
        scratch_types=[
            pltpu.VMEM((cpt, CHUNK), jnp.int32),
            pltpu.VMEM((cpt, CHUNK), jnp.int32),
            pltpu.VMEM((CHUNK, f), jnp.float32),
            pltpu.VMEM_SHARED((np_, f), jnp.float32),
            pltpu.SemaphoreType.DMA,
        ],
        compiler_params=pltpu.CompilerParams(use_tc_tiling_on_sc=False),
    )
    return k(hs, src2d, dst2d, zeros_f)


_BLK = 1024


def _prep_body(g, fo, deg2_ref, batch_ref, lmi_ref, x_ref, wall_ref,
               dinv_ref, c_ref, diag_ref, y_ref, hs_ref, acc_ref):
    deg = deg2_ref[0] + deg2_ref[1]
    dinv = jnp.where(deg > 0.0, lax.rsqrt(jnp.where(deg > 0.0, deg, 1.0)), 0.0)
    b = batch_ref[...]
    onehot = (b == lax.broadcasted_iota(jnp.int32, (b.shape[0], g), 1)).astype(
        jnp.float32
    )
    lam2 = onehot @ lmi_ref[...]
    dinv_ref[...] = dinv
    diag_ref[...] = lam2 - 1.0
    c_ref[...] = lam2 * dinv
    z = jnp.dot(x_ref[...], wall_ref[...], preferred_element_type=jnp.float32)
    acc_ref[...] = z[:, :fo]
    y = z[:, fo:]
    y_ref[...] = y
    hs_ref[...] = (lam2 * dinv) * y


def _tc_prep(deg2, batch_pad, lmi, x, wall, fo, np_):
    g = lmi.shape[0]
    d = x.shape[1]
    fw = wall.shape[1] - fo
    nb = np_ // _BLK
    return pl.pallas_call(
        functools.partial(_prep_body, g, fo),
        grid=(nb,),
        in_specs=[
            pl.BlockSpec((NC, _BLK, 1), lambda i: (0, i, 0)),
            pl.BlockSpec((_BLK, 1), lambda i: (i, 0)),
            pl.BlockSpec((g, 1), lambda i: (0, 0)),
            pl.BlockSpec((_BLK, d), lambda i: (i, 0)),
            pl.BlockSpec((d, fo + fw), lambda i: (0, 0)),
        ],
        out_specs=[
            pl.BlockSpec((_BLK, 1), lambda i: (i, 0)),
            pl.BlockSpec((_BLK, 1), lambda i: (i, 0)),
            pl.BlockSpec((_BLK, 1), lambda i: (i, 0)),
            pl.BlockSpec((_BLK, fw), lambda i: (i, 0)),
            pl.BlockSpec((_BLK, fw), lambda i: (i, 0)),
            pl.BlockSpec((_BLK, fo), lambda i: (i, 0)),
        ],
        out_shape=[
            jax.ShapeDtypeStruct((np_, 1), jnp.float32),
            jax.ShapeDtypeStruct((np_, 1), jnp.float32),
            jax.ShapeDtypeStruct((np_, 1), jnp.float32),
            jax.ShapeDtypeStruct((np_, fw), jnp.float32),
            jax.ShapeDtypeStruct((np_, fw), jnp.float32),
            jax.ShapeDtypeStruct((np_, fo), jnp.float32),
        ],
    )(deg2, batch_pad, lmi, x, wall)


def _chain_body(alpha, beta, fo, parts_ref, tp_ref, tp2_ref, dinv_ref,
                c_ref, diag_ref, accin_ref, accout_ref, tn_ref, hs_ref,
                tp2n_ref):
    agg = parts_ref[0] + parts_ref[1]
    tp = tp_ref[...]
    lh = diag_ref[...] * tp - dinv_ref[...] * agg
    t = alpha * lh - beta * tp2_ref[...] if beta else alpha * lh
    accout_ref[...] = accin_ref[...] + t[:, :fo]
    tn = t[:, fo:]
    tn_ref[...] = tn
    hs_ref[...] = c_ref[...] * tn
    tp2n_ref[...] = tp[:, fo:]


def _tc_chain(parts, tp, tp2, dinv, cvec, diag, accin, alpha, beta, fo, np_):
    f = tp.shape[1]
    fn = f - fo
    nb = np_ // _BLK
    return pl.pallas_call(
        functools.partial(_chain_body, alpha, beta, fo),
        grid=(nb,),
        in_specs=[
            pl.BlockSpec((NC, _BLK, f), lambda i: (0, i, 0)),
            pl.BlockSpec((_BLK, f), lambda i: (i, 0)),
            pl.BlockSpec((_BLK, f), lambda i: (i, 0)),
            pl.BlockSpec((_BLK, 1), lambda i: (i, 0)),
            pl.BlockSpec((_BLK, 1), lambda i: (i, 0)),
            pl.BlockSpec((_BLK, 1), lambda i: (i, 0)),
            pl.BlockSpec((_BLK, fo), lambda i: (i, 0)),
        ],
        out_specs=[
            pl.BlockSpec((_BLK, fo), lambda i: (i, 0)),
            pl.BlockSpec((_BLK, fn), lambda i: (i, 0)),
            pl.BlockSpec((_BLK, fn), lambda i: (i, 0)),
            pl.BlockSpec((_BLK, fn), lambda i: (i, 0)),
        ],
        out_shape=[
            jax.ShapeDtypeStruct((np_, fo), jnp.float32),
            jax.ShapeDtypeStruct((np_, fn), jnp.float32),
            jax.ShapeDtypeStruct((np_, fn), jnp.float32),
            jax.ShapeDtypeStruct((np_, fn), jnp.float32),
        ],
    )(parts, tp, tp2, dinv, cvec, diag, accin)


def _step_body(alpha, beta, parts_ref, tp_ref, tp2_ref, dinv_ref,
               c_ref, diag_ref, wk_ref, accin_ref, tx_ref, hs_ref, accout_ref):
    agg = parts_ref[0] + parts_ref[1]
    tp = tp_ref[...]
    lh = diag_ref[...] * tp - dinv_ref[...] * agg
    tx = alpha * lh - beta * tp2_ref[...] if beta else alpha * lh
    tx_ref[...] = tx
    hs_ref[...] = c_ref[...] * tx
    accout_ref[...] = accin_ref[...] + jnp.dot(
        tx, wk_ref[...], preferred_element_type=jnp.float32
    )


def _tc_step(parts, tp, tp2, dinv, cvec, diag, wk, accin, alpha, beta, np_):
    f = tp.shape[1]
    fo = wk.shape[1]
    nb = np_ // _BLK
    return pl.pallas_call(
        functools.partial(_step_body, alpha, beta),
        grid=(nb,),
        in_specs=[
            pl.BlockSpec((NC, _BLK, f), lambda i: (0, i, 0)),
            pl.BlockSpec((_BLK, f), lambda i: (i, 0)),
            pl.BlockSpec((_BLK, f), lambda i: (i, 0)),
            pl.BlockSpec((_BLK, 1), lambda i: (i, 0)),
            pl.BlockSpec((_BLK, 1), lambda i: (i, 0)),
            pl.BlockSpec((_BLK, 1), lambda i: (i, 0)),
            pl.BlockSpec((f, fo), lambda i: (0, 0)),
            pl.BlockSpec((_BLK, fo), lambda i: (i, 0)),
        ],
        out_specs=[
            pl.BlockSpec((_BLK, f), lambda i: (i, 0)),
            pl.BlockSpec((_BLK, f), lambda i: (i, 0)),
            pl.BlockSpec((_BLK, fo), lambda i: (i, 0)),
        ],
        out_shape=[
            jax.ShapeDtypeStruct((np_, f), jnp.float32),
            jax.ShapeDtypeStruct((np_, f), jnp.float32),
            jax.ShapeDtypeStruct((np_, fo), jnp.float32),
        ],
    )(parts, tp, tp2, dinv, cvec, diag, wk, accin)


def _fin_body(project, parts_ref, tp_ref, tp2_ref, dinv_ref, c_ref, diag_ref,
              *rest):
    if project:
        wk_ref, accin_ref, bias_ref, wn_ref, h_ref, hs_ref, accn_ref = rest
    else:
        accin_ref, bias_ref, wn_ref, h_ref, hs_ref, accn_ref = rest
    agg = parts_ref[0] + parts_ref[1]
    lh = diag_ref[...] * tp_ref[...] - dinv_ref[...] * agg
    tx = 2.0 * lh - tp2_ref[...]
    if project:
        o = accin_ref[...] + jnp.dot(
            tx, wk_ref[...], preferred_element_type=jnp.float32
        )
    else:
        o = accin_ref[...] + tx
    h = jnp.maximum(o + bias_ref[...], 0.0)
    h_ref[...] = h
    hs_ref[...] = c_ref[...] * h
    accn_ref[...] = jnp.dot(h, wn_ref[...], preferred_element_type=jnp.float32)


def _tc_finish(parts, tp, tp2, dinv, cvec, diag, wk, accin, bias, wnext, np_):
    f = tp.shape[1]
    fo = accin.shape[1]
    fn = wnext.shape[1]
    nb = np_ // _BLK
    project = wk is not None
    in_specs = [
        pl.BlockSpec((NC, _BLK, f), lambda i: (0, i, 0)),
        pl.BlockSpec((_BLK, f), lambda i: (i, 0)),
        pl.BlockSpec((_BLK, f), lambda i: (i, 0)),
        pl.BlockSpec((_BLK, 1), lambda i: (i, 0)),
        pl.BlockSpec((_BLK, 1), lambda i: (i, 0)),
        pl.BlockSpec((_BLK, 1), lambda i: (i, 0)),
    ]
    args = [parts, tp, tp2, dinv, cvec, diag]
    if project:
        in_specs.append(pl.BlockSpec((f, fo), lambda i: (0, 0)))
        args.append(wk)
    in_specs += [
        pl.BlockSpec((_BLK, fo), lambda i: (i, 0)),
        pl.BlockSpec((1, fo), lambda i: (0, 0)),
        pl.BlockSpec((fo, fn), lambda i: (0, 0)),
    ]
    args += [accin, bias, wnext]
    return pl.pallas_call(
        functools.partial(_fin_body, project),
        grid=(nb,),
        in_specs=in_specs,
        out_specs=[
            pl.BlockSpec((_BLK, fo), lambda i: (i, 0)),
            pl.BlockSpec((_BLK, fo), lambda i: (i, 0)),
            pl.BlockSpec((_BLK, fn), lambda i: (i, 0)),
        ],
        out_shape=[
            jax.ShapeDtypeStruct((np_, fo), jnp.float32),
            jax.ShapeDtypeStruct((np_, fo), jnp.float32),
            jax.ShapeDtypeStruct((np_, fn), jnp.float32),
        ],
    )(*args)


def _pool_body(g, nb, h_ref, batch_ref, a1w_ref, a1b_ref, a2w_ref, a2b_ref,
               out_ref, acc_ref):
    i = pl.program_id(0)

    @pl.when(i == 0)
    def _init():
        acc_ref[...] = jnp.full_like(acc_ref, -jnp.inf)

    h = h_ref[...]
    b = batch_ref[...]
    for gg in range(g):
        sel = jnp.where(b == gg, h, -jnp.inf)
        acc_ref[gg, :] = jnp.maximum(acc_ref[gg, :], jnp.max(sel, axis=0))

    @pl.when(i == nb - 1)
    def _fin():
        gmax = acc_ref[...]
        gmax = jnp.where(jnp.isfinite(gmax), gmax, 0.0)
        z = jnp.maximum(
            jnp.dot(gmax, a1w_ref[...], preferred_element_type=jnp.float32)
            + a1b_ref[...],
            0.0,
        )
        out_ref[...] = (
            jnp.dot(z, a2w_ref[...], preferred_element_type=jnp.float32)
            + a2b_ref[...]
        )


def _tc_pool(h3, batch_pad, a1w, a1b, a2w, a2b, g, np_):
    f = h3.shape[1]
    nb = np_ // _BLK
    return pl.pallas_call(
        functools.partial(_pool_body, g, nb),
        grid=(nb,),
        in_specs=[
            pl.BlockSpec((_BLK, f), lambda i: (i, 0)),
            pl.BlockSpec((_BLK, 1), lambda i: (i, 0)),
            pl.BlockSpec((f, 16), lambda i: (0, 0)),
            pl.BlockSpec((1, 16), lambda i: (0, 0)),
            pl.BlockSpec((16, 1), lambda i: (0, 0)),
            pl.BlockSpec((1, 1), lambda i: (0, 0)),
        ],
        out_specs=pl.BlockSpec((g, 1), lambda i: (0, 0)),
        out_shape=jax.ShapeDtypeStruct((g, 1), jnp.float32),
        scratch_shapes=[pltpu.VMEM((g, f), jnp.float32)],
    )(h3, batch_pad, a1w, a1b, a2w, a2b)


def kernel(x, edge_index, batch, lmax, W1, b1, W2, b2, W3, b3, A1w, A1b, A2w, A2b):
    n, d = x.shape
    e = edge_index.shape[1]
    g = lmax.shape[0]
    s_order = W1.shape[0]
    fo1 = W1.shape[2]

    np_ = ((n + 16 + _BLK - 1) // _BLK) * _BLK
    unit = CHUNK * NC * NS * 8
    ecp = ((e + unit - 1) // unit) * NC * NS * 8
    pad_e = ecp * CHUNK - e

    src = edge_index[0]
    dst = edge_index[1]
    pad_idx = n + (jnp.arange(pad_e, dtype=jnp.int32) % 16)
    src2d = jnp.concatenate([src, pad_idx]).reshape(ecp, CHUNK)
    dst2d = jnp.concatenate([dst, pad_idx]).reshape(ecp, CHUNK)

    x_pad = jnp.zeros((np_, d), jnp.float32).at[:n].set(x)
    batch_pad = jnp.full((np_, 1), g, jnp.int32).at[:n, 0].set(batch)
    lmi = (2.0 / lmax).reshape(g, 1)

    widths = {16, W2.shape[1], W3.shape[1]}
    widths.update((s_order - 1 - j) * fo1 for j in range(s_order - 1))
    zeros = {f: jnp.zeros((np_, f), jnp.float32) for f in widths}

    ones16 = jnp.ones((np_, 16), jnp.float32)
    deg16 = _sc_propagate(ones16, src2d, src2d, zeros[16], np_)
    deg2 = deg16[:, :, :1]

    wall = jnp.concatenate([W1[k] for k in range(s_order)], axis=1)
    dinv, cvec, diag, tp, hs, acc = _tc_prep(
        deg2, batch_pad, lmi, x_pad, wall, fo1, np_
    )
    tp2 = tp
    for j in range(1, s_order):
        fw = (s_order - j) * fo1
        parts = _sc_propagate(hs, src2d, dst2d, zeros[fw], np_)
        alpha, beta = (1.0, 0.0) if j == 1 else (2.0, 1.0)
        if j < s_order - 1:
            acc, tn, hs, tp2n = _tc_chain(
                parts, tp, tp2, dinv, cvec, diag, acc, alpha, beta, fo1, np_
            )
            tp, tp2 = tn, tp2n
        else:
            h, hs, acc = _tc_finish(
                parts, tp, tp2, dinv, cvec, diag, None, acc,
                b1.reshape(1, -1), W2[0], np_,
            )

    for W, b, wnext in ((W2, b2, W3[0]), (W3, b3, None)):
        f_in = W.shape[1]
        tp2 = h
        tp = None
        for k in range(1, s_order):
            parts = _sc_propagate(hs, src2d, dst2d, zeros[f_in], np_)
            if k == 1:
                tp, hs, acc = _tc_step(
                    parts, tp2, tp2, dinv, cvec, diag, W[1], acc, 1.0, 0.0, np_
                )
            elif k < s_order - 1:
                tx, hs, acc = _tc_step(
                    parts, tp, tp2, dinv, cvec, diag, W[k], acc, 2.0, 1.0, np_
                )
                tp2, tp = tp, tx
            else:
                wn = wnext if wnext is not None else jnp.zeros(
                    (W.shape[2], 8), jnp.float32
                )
                h, hs, acc = _tc_finish(
                    parts, tp, tp2, dinv, cvec, diag, W[k], acc,
                    b.reshape(1, -1), wn, np_,
                )

    return _tc_pool(h, batch_pad, A1w, A1b.reshape(1, -1), A2w,
                    A2b.reshape(1, -1), g, np_)


# speedup vs baseline: 1.2795x; 1.1673x over previous
import functools

import jax
import jax.numpy as jnp
from jax import lax
from jax.experimental import pallas as pl
from jax.experimental.pallas import tpu as pltpu
from jax.experimental.pallas import tpu_sc as plsc

NC = 2
NS = 16
CHUNK = 128


@functools.cache
def _mesh():
    return plsc.VectorSubcoreMesh(
        core_axis_name="c", subcore_axis_name="s", num_cores=NC, num_subcores=NS
    )


_SB = 16


def _prop_body(cpt, np_, f, hs_hbm, src_hbm, dst_hbm, zeros_hbm, out_hbm,
               srcv, dstv, rows0, rows1, acc, sem0, sem1):
    c = lax.axis_index("c")
    s = lax.axis_index("s")
    wid = c * NS + s
    rows = np_ // NS
    pltpu.sync_copy(zeros_hbm.at[pl.ds(s * rows, rows)], acc.at[pl.ds(s * rows, rows)])
    pltpu.sync_copy(src_hbm.at[pl.ds(wid * cpt, cpt)], srcv)
    pltpu.sync_copy(dst_hbm.at[pl.ds(wid * cpt, _SB)], dstv)
    plsc.subcore_barrier()

    pltpu.async_copy(hs_hbm.at[srcv.at[0]], rows0, sem0)

    def body(i, carry):
        j = 2 * i

        @pl.when(j % _SB == 0)
        def _reload():
            pltpu.sync_copy(dst_hbm.at[pl.ds(wid * cpt + j, _SB)], dstv)

        pltpu.make_async_copy(hs_hbm.at[srcv.at[j]], rows0, sem0).wait()
        pltpu.async_copy(hs_hbm.at[srcv.at[j + 1]], rows1, sem1)
        pltpu.sync_copy(rows0, acc.at[dstv.at[j % _SB]], add=True)
        pltpu.make_async_copy(hs_hbm.at[srcv.at[j + 1]], rows1, sem1).wait()

        @pl.when(j + 2 < cpt)
        def _next():
            pltpu.async_copy(hs_hbm.at[srcv.at[j + 2]], rows0, sem0)

        pltpu.sync_copy(rows1, acc.at[dstv.at[j % _SB + 1]], add=True)
        return carry

    lax.fori_loop(0, cpt // 2, body, 0)
    plsc.subcore_barrier()
    pltpu.sync_copy(acc.at[pl.ds(s * rows, rows)], out_hbm.at[c, pl.ds(s * rows, rows)])


def _sc_propagate(hs, src2d, dst2d, zeros_f, np_):
    f = hs.shape[1]
    cpt = src2d.shape[0] // (NC * NS)
    k = pl.kernel(
        functools.partial(_prop_body, cpt, np_, f),
        out_type=jax.ShapeDtypeStruct((NC, np_, f), jnp.float32),
        mesh=_mesh(),
        scratch_types=[
            pltpu.VMEM((cpt, CHUNK), jnp.int32),
            pltpu.VMEM((_SB, CHUNK), jnp.int32),
            pltpu.VMEM((CHUNK, f), jnp.float32),
            pltpu.VMEM((CHUNK, f), jnp.float32),
            pltpu.VMEM_SHARED((np_, f), jnp.float32),
            pltpu.SemaphoreType.DMA,
            pltpu.SemaphoreType.DMA,
        ],
        compiler_params=pltpu.CompilerParams(use_tc_tiling_on_sc=False),
    )
    return k(hs, src2d, dst2d, zeros_f)


_BLK = 1024


def _prep_body(g, fo, deg2_ref, batch_ref, lmi_ref, x_ref, wall_ref,
               dinv_ref, c_ref, diag_ref, y_ref, hs_ref, acc_ref):
    deg = deg2_ref[0] + deg2_ref[1]
    dinv = jnp.where(deg > 0.0, lax.rsqrt(jnp.where(deg > 0.0, deg, 1.0)), 0.0)
    b = batch_ref[...]
    onehot = (b == lax.broadcasted_iota(jnp.int32, (b.shape[0], g), 1)).astype(
        jnp.float32
    )
    lam2 = onehot @ lmi_ref[...]
    dinv_ref[...] = dinv
    diag_ref[...] = lam2 - 1.0
    c_ref[...] = lam2 * dinv
    z = jnp.dot(x_ref[...], wall_ref[...], preferred_element_type=jnp.float32)
    acc_ref[...] = z[:, :fo]
    y = z[:, fo:]
    y_ref[...] = y
    hs_ref[...] = (lam2 * dinv) * y


def _tc_prep(deg2, batch_pad, lmi, x, wall, fo, np_):
    g = lmi.shape[0]
    d = x.shape[1]
    fw = wall.shape[1] - fo
    nb = np_ // _BLK
    return pl.pallas_call(
        functools.partial(_prep_body, g, fo),
        grid=(nb,),
        in_specs=[
            pl.BlockSpec((NC, _BLK, 1), lambda i: (0, i, 0)),
            pl.BlockSpec((_BLK, 1), lambda i: (i, 0)),
            pl.BlockSpec((g, 1), lambda i: (0, 0)),
            pl.BlockSpec((_BLK, d), lambda i: (i, 0)),
            pl.BlockSpec((d, fo + fw), lambda i: (0, 0)),
        ],
        out_specs=[
            pl.BlockSpec((_BLK, 1), lambda i: (i, 0)),
            pl.BlockSpec((_BLK, 1), lambda i: (i, 0)),
            pl.BlockSpec((_BLK, 1), lambda i: (i, 0)),
            pl.BlockSpec((_BLK, fw), lambda i: (i, 0)),
            pl.BlockSpec((_BLK, fw), lambda i: (i, 0)),
            pl.BlockSpec((_BLK, fo), lambda i: (i, 0)),
        ],
        out_shape=[
            jax.ShapeDtypeStruct((np_, 1), jnp.float32),
            jax.ShapeDtypeStruct((np_, 1), jnp.float32),
            jax.ShapeDtypeStruct((np_, 1), jnp.float32),
            jax.ShapeDtypeStruct((np_, fw), jnp.float32),
            jax.ShapeDtypeStruct((np_, fw), jnp.float32),
            jax.ShapeDtypeStruct((np_, fo), jnp.float32),
        ],
    )(deg2, batch_pad, lmi, x, wall)


def _chain_body(alpha, beta, fo, parts_ref, tp_ref, tp2_ref, dinv_ref,
                c_ref, diag_ref, accin_ref, accout_ref, tn_ref, hs_ref,
                tp2n_ref):
    agg = parts_ref[0] + parts_ref[1]
    tp = tp_ref[...]
    lh = diag_ref[...] * tp - dinv_ref[...] * agg
    t = alpha * lh - beta * tp2_ref[...] if beta else alpha * lh
    accout_ref[...] = accin_ref[...] + t[:, :fo]
    tn = t[:, fo:]
    tn_ref[...] = tn
    hs_ref[...] = c_ref[...] * tn
    tp2n_ref[...] = tp[:, fo:]


def _tc_chain(parts, tp, tp2, dinv, cvec, diag, accin, alpha, beta, fo, np_):
    f = tp.shape[1]
    fn = f - fo
    nb = np_ // _BLK
    return pl.pallas_call(
        functools.partial(_chain_body, alpha, beta, fo),
        grid=(nb,),
        in_specs=[
            pl.BlockSpec((NC, _BLK, f), lambda i: (0, i, 0)),
            pl.BlockSpec((_BLK, f), lambda i: (i, 0)),
            pl.BlockSpec((_BLK, f), lambda i: (i, 0)),
            pl.BlockSpec((_BLK, 1), lambda i: (i, 0)),
            pl.BlockSpec((_BLK, 1), lambda i: (i, 0)),
            pl.BlockSpec((_BLK, 1), lambda i: (i, 0)),
            pl.BlockSpec((_BLK, fo), lambda i: (i, 0)),
        ],
        out_specs=[
            pl.BlockSpec((_BLK, fo), lambda i: (i, 0)),
            pl.BlockSpec((_BLK, fn), lambda i: (i, 0)),
            pl.BlockSpec((_BLK, fn), lambda i: (i, 0)),
            pl.BlockSpec((_BLK, fn), lambda i: (i, 0)),
        ],
        out_shape=[
            jax.ShapeDtypeStruct((np_, fo), jnp.float32),
            jax.ShapeDtypeStruct((np_, fn), jnp.float32),
            jax.ShapeDtypeStruct((np_, fn), jnp.float32),
            jax.ShapeDtypeStruct((np_, fn), jnp.float32),
        ],
    )(parts, tp, tp2, dinv, cvec, diag, accin)


def _step_body(alpha, beta, parts_ref, tp_ref, tp2_ref, dinv_ref,
               c_ref, diag_ref, wk_ref, accin_ref, tx_ref, hs_ref, accout_ref):
    agg = parts_ref[0] + parts_ref[1]
    tp = tp_ref[...]
    lh = diag_ref[...] * tp - dinv_ref[...] * agg
    tx = alpha * lh - beta * tp2_ref[...] if beta else alpha * lh
    tx_ref[...] = tx
    hs_ref[...] = c_ref[...] * tx
    accout_ref[...] = accin_ref[...] + jnp.dot(
        tx, wk_ref[...], preferred_element_type=jnp.float32
    )


def _tc_step(parts, tp, tp2, dinv, cvec, diag, wk, accin, alpha, beta, np_):
    f = tp.shape[1]
    fo = wk.shape[1]
    nb = np_ // _BLK
    return pl.pallas_call(
        functools.partial(_step_body, alpha, beta),
        grid=(nb,),
        in_specs=[
            pl.BlockSpec((NC, _BLK, f), lambda i: (0, i, 0)),
            pl.BlockSpec((_BLK, f), lambda i: (i, 0)),
            pl.BlockSpec((_BLK, f), lambda i: (i, 0)),
            pl.BlockSpec((_BLK, 1), lambda i: (i, 0)),
            pl.BlockSpec((_BLK, 1), lambda i: (i, 0)),
            pl.BlockSpec((_BLK, 1), lambda i: (i, 0)),
            pl.BlockSpec((f, fo), lambda i: (0, 0)),
            pl.BlockSpec((_BLK, fo), lambda i: (i, 0)),
        ],
        out_specs=[
            pl.BlockSpec((_BLK, f), lambda i: (i, 0)),
            pl.BlockSpec((_BLK, f), lambda i: (i, 0)),
            pl.BlockSpec((_BLK, fo), lambda i: (i, 0)),
        ],
        out_shape=[
            jax.ShapeDtypeStruct((np_, f), jnp.float32),
            jax.ShapeDtypeStruct((np_, f), jnp.float32),
            jax.ShapeDtypeStruct((np_, fo), jnp.float32),
        ],
    )(parts, tp, tp2, dinv, cvec, diag, wk, accin)


def _fin_body(project, parts_ref, tp_ref, tp2_ref, dinv_ref, c_ref, diag_ref,
              *rest):
    if project:
        wk_ref, accin_ref, bias_ref, wn_ref, h_ref, hs_ref, accn_ref = rest
    else:
        accin_ref, bias_ref, wn_ref, h_ref, hs_ref, accn_ref = rest
    agg = parts_ref[0] + parts_ref[1]
    lh = diag_ref[...] * tp_ref[...] - dinv_ref[...] * agg
    tx = 2.0 * lh - tp2_ref[...]
    if project:
        o = accin_ref[...] + jnp.dot(
            tx, wk_ref[...], preferred_element_type=jnp.float32
        )
    else:
        o = accin_ref[...] + tx
    h = jnp.maximum(o + bias_ref[...], 0.0)
    h_ref[...] = h
    hs_ref[...] = c_ref[...] * h
    accn_ref[...] = jnp.dot(h, wn_ref[...], preferred_element_type=jnp.float32)


def _tc_finish(parts, tp, tp2, dinv, cvec, diag, wk, accin, bias, wnext, np_):
    f = tp.shape[1]
    fo = accin.shape[1]
    fn = wnext.shape[1]
    nb = np_ // _BLK
    project = wk is not None
    in_specs = [
        pl.BlockSpec((NC, _BLK, f), lambda i: (0, i, 0)),
        pl.BlockSpec((_BLK, f), lambda i: (i, 0)),
        pl.BlockSpec((_BLK, f), lambda i: (i, 0)),
        pl.BlockSpec((_BLK, 1), lambda i: (i, 0)),
        pl.BlockSpec((_BLK, 1), lambda i: (i, 0)),
        pl.BlockSpec((_BLK, 1), lambda i: (i, 0)),
    ]
    args = [parts, tp, tp2, dinv, cvec, diag]
    if project:
        in_specs.append(pl.BlockSpec((f, fo), lambda i: (0, 0)))
        args.append(wk)
    in_specs += [
        pl.BlockSpec((_BLK, fo), lambda i: (i, 0)),
        pl.BlockSpec((1, fo), lambda i: (0, 0)),
        pl.BlockSpec((fo, fn), lambda i: (0, 0)),
    ]
    args += [accin, bias, wnext]
    return pl.pallas_call(
        functools.partial(_fin_body, project),
        grid=(nb,),
        in_specs=in_specs,
        out_specs=[
            pl.BlockSpec((_BLK, fo), lambda i: (i, 0)),
            pl.BlockSpec((_BLK, fo), lambda i: (i, 0)),
            pl.BlockSpec((_BLK, fn), lambda i: (i, 0)),
        ],
        out_shape=[
            jax.ShapeDtypeStruct((np_, fo), jnp.float32),
            jax.ShapeDtypeStruct((np_, fo), jnp.float32),
            jax.ShapeDtypeStruct((np_, fn), jnp.float32),
        ],
    )(*args)


def _pool_body(g, nb, h_ref, batch_ref, a1w_ref, a1b_ref, a2w_ref, a2b_ref,
               out_ref, acc_ref):
    i = pl.program_id(0)

    @pl.when(i == 0)
    def _init():
        acc_ref[...] = jnp.full_like(acc_ref, -jnp.inf)

    h = h_ref[...]
    b = batch_ref[...]
    for gg in range(g):
        sel = jnp.where(b == gg, h, -jnp.inf)
        acc_ref[gg, :] = jnp.maximum(acc_ref[gg, :], jnp.max(sel, axis=0))

    @pl.when(i == nb - 1)
    def _fin():
        gmax = acc_ref[...]
        gmax = jnp.where(jnp.isfinite(gmax), gmax, 0.0)
        z = jnp.maximum(
            jnp.dot(gmax, a1w_ref[...], preferred_element_type=jnp.float32)
            + a1b_ref[...],
            0.0,
        )
        out_ref[...] = (
            jnp.dot(z, a2w_ref[...], preferred_element_type=jnp.float32)
            + a2b_ref[...]
        )


def _tc_pool(h3, batch_pad, a1w, a1b, a2w, a2b, g, np_):
    f = h3.shape[1]
    nb = np_ // _BLK
    return pl.pallas_call(
        functools.partial(_pool_body, g, nb),
        grid=(nb,),
        in_specs=[
            pl.BlockSpec((_BLK, f), lambda i: (i, 0)),
            pl.BlockSpec((_BLK, 1), lambda i: (i, 0)),
            pl.BlockSpec((f, 16), lambda i: (0, 0)),
            pl.BlockSpec((1, 16), lambda i: (0, 0)),
            pl.BlockSpec((16, 1), lambda i: (0, 0)),
            pl.BlockSpec((1, 1), lambda i: (0, 0)),
        ],
        out_specs=pl.BlockSpec((g, 1), lambda i: (0, 0)),
        out_shape=jax.ShapeDtypeStruct((g, 1), jnp.float32),
        scratch_shapes=[pltpu.VMEM((g, f), jnp.float32)],
    )(h3, batch_pad, a1w, a1b, a2w, a2b)


def kernel(x, edge_index, batch, lmax, W1, b1, W2, b2, W3, b3, A1w, A1b, A2w, A2b):
    n, d = x.shape
    e = edge_index.shape[1]
    g = lmax.shape[0]
    s_order = W1.shape[0]
    fo1 = W1.shape[2]

    np_ = ((n + 16 + _BLK - 1) // _BLK) * _BLK
    unit = CHUNK * NC * NS * 8
    ecp = ((e + unit - 1) // unit) * NC * NS * 8
    pad_e = ecp * CHUNK - e

    src = edge_index[0]
    dst = edge_index[1]
    pad_idx = n + (jnp.arange(pad_e, dtype=jnp.int32) % 16)
    src2d = jnp.concatenate([src, pad_idx]).reshape(ecp, CHUNK)
    dst2d = jnp.concatenate([dst, pad_idx]).reshape(ecp, CHUNK)

    x_pad = jnp.zeros((np_, d), jnp.float32).at[:n].set(x)
    batch_pad = jnp.full((np_, 1), g, jnp.int32).at[:n, 0].set(batch)
    lmi = (2.0 / lmax).reshape(g, 1)

    widths = {16, W2.shape[1], W3.shape[1]}
    widths.update((s_order - 1 - j) * fo1 for j in range(s_order - 1))
    zeros = {f: jnp.zeros((np_, f), jnp.float32) for f in widths}

    ones16 = jnp.ones((np_, 16), jnp.float32)
    deg16 = _sc_propagate(ones16, src2d, src2d, zeros[16], np_)
    deg2 = deg16[:, :, :1]

    wall = jnp.concatenate([W1[k] for k in range(s_order)], axis=1)
    dinv, cvec, diag, tp, hs, acc = _tc_prep(
        deg2, batch_pad, lmi, x_pad, wall, fo1, np_
    )
    tp2 = tp
    for j in range(1, s_order):
        fw = (s_order - j) * fo1
        parts = _sc_propagate(hs, src2d, dst2d, zeros[fw], np_)
        alpha, beta = (1.0, 0.0) if j == 1 else (2.0, 1.0)
        if j < s_order - 1:
            acc, tn, hs, tp2n = _tc_chain(
                parts, tp, tp2, dinv, cvec, diag, acc, alpha, beta, fo1, np_
            )
            tp, tp2 = tn, tp2n
        else:
            h, hs, acc = _tc_finish(
                parts, tp, tp2, dinv, cvec, diag, None, acc,
                b1.reshape(1, -1), W2[0], np_,
            )

    for W, b, wnext in ((W2, b2, W3[0]), (W3, b3, None)):
        f_in = W.shape[1]
        tp2 = h
        tp = None
        for k in range(1, s_order):
            parts = _sc_propagate(hs, src2d, dst2d, zeros[f_in], np_)
            if k == 1:
                tp, hs, acc = _tc_step(
                    parts, tp2, tp2, dinv, cvec, diag, W[1], acc, 1.0, 0.0, np_
                )
            elif k < s_order - 1:
                tx, hs, acc = _tc_step(
                    parts, tp, tp2, dinv, cvec, diag, W[k], acc, 2.0, 1.0, np_
                )
                tp2, tp = tp, tx
            else:
                wn = wnext if wnext is not None else jnp.zeros(
                    (W.shape[2], 8), jnp.float32
                )
                h, hs, acc = _tc_finish(
                    parts, tp, tp2, dinv, cvec, diag, W[k], acc,
                    b.reshape(1, -1), wn, np_,
                )

    return _tc_pool(h, batch_pad, A1w, A1b.reshape(1, -1), A2w,
                    A2b.reshape(1, -1), g, np_)


# speedup vs baseline: 1.3939x; 1.0894x over previous
import functools

import jax
import jax.numpy as jnp
from jax import lax
from jax.experimental import pallas as pl
from jax.experimental.pallas import tpu as pltpu
from jax.experimental.pallas import tpu_sc as plsc

NC = 2
NS = 16
CHUNK = 128


@functools.cache
def _mesh():
    return plsc.VectorSubcoreMesh(
        core_axis_name="c", subcore_axis_name="s", num_cores=NC, num_subcores=NS
    )


_SB = 16


def _prop_body(cpt, np_, f, hs_hbm, src_hbm, dst_hbm, zeros_hbm, out_hbm,
               srcv, dstv, rows0, rows1, acc, sem0, sem1, ssem0, ssem1):
    c = lax.axis_index("c")
    s = lax.axis_index("s")
    wid = c * NS + s
    rows = np_ // NS
    pltpu.sync_copy(zeros_hbm.at[pl.ds(s * rows, rows)], acc.at[pl.ds(s * rows, rows)])
    pltpu.sync_copy(src_hbm.at[pl.ds(wid * cpt, cpt)], srcv)
    pltpu.sync_copy(dst_hbm.at[pl.ds(wid * cpt, _SB)], dstv)
    plsc.subcore_barrier()

    pltpu.async_copy(hs_hbm.at[srcv.at[0]], rows0, sem0)
    pltpu.async_copy(hs_hbm.at[srcv.at[1]], rows1, sem1)

    def body(i, carry):
        j = 2 * i

        @pl.when(j % _SB == 0)
        def _reload():
            pltpu.sync_copy(dst_hbm.at[pl.ds(wid * cpt + j, _SB)], dstv)

        pltpu.make_async_copy(hs_hbm.at[srcv.at[j]], rows0, sem0).wait()
        pltpu.async_copy(rows0, acc.at[dstv.at[j % _SB]], ssem0, add=True)
        pltpu.make_async_copy(hs_hbm.at[srcv.at[j + 1]], rows1, sem1).wait()
        pltpu.async_copy(rows1, acc.at[dstv.at[j % _SB + 1]], ssem1, add=True)
        pltpu.make_async_copy(rows0, acc.at[dstv.at[j % _SB]], ssem0).wait()

        @pl.when(j + 2 < cpt)
        def _n0():
            pltpu.async_copy(hs_hbm.at[srcv.at[j + 2]], rows0, sem0)

        pltpu.make_async_copy(rows1, acc.at[dstv.at[j % _SB + 1]], ssem1).wait()

        @pl.when(j + 3 < cpt)
        def _n1():
            pltpu.async_copy(hs_hbm.at[srcv.at[j + 3]], rows1, sem1)

        return carry

    lax.fori_loop(0, cpt // 2, body, 0)
    plsc.subcore_barrier()
    pltpu.sync_copy(acc.at[pl.ds(s * rows, rows)], out_hbm.at[c, pl.ds(s * rows, rows)])


def _sc_propagate(hs, src2d, dst2d, zeros_f, np_):
    f = hs.shape[1]
    cpt = src2d.shape[0] // (NC * NS)
    k = pl.kernel(
        functools.partial(_prop_body, cpt, np_, f),
        out_type=jax.ShapeDtypeStruct((NC, np_, f), jnp.float32),
        mesh=_mesh(),
        scratch_types=[
            pltpu.VMEM((cpt, CHUNK), jnp.int32),
            pltpu.VMEM((_SB, CHUNK), jnp.int32),
            pltpu.VMEM((CHUNK, f), jnp.float32),
            pltpu.VMEM((CHUNK, f), jnp.float32),
            pltpu.VMEM_SHARED((np_, f), jnp.float32),
            pltpu.SemaphoreType.DMA,
            pltpu.SemaphoreType.DMA,
            pltpu.SemaphoreType.DMA,
            pltpu.SemaphoreType.DMA,
        ],
        compiler_params=pltpu.CompilerParams(use_tc_tiling_on_sc=False),
    )
    return k(hs, src2d, dst2d, zeros_f)


_BLK = 1024


def _prep_body(g, fo, deg2_ref, batch_ref, lmi_ref, x_ref, wall_ref,
               dinv_ref, c_ref, diag_ref, y_ref, hs_ref, acc_ref):
    deg = deg2_ref[0] + deg2_ref[1]
    dinv = jnp.where(deg > 0.0, lax.rsqrt(jnp.where(deg > 0.0, deg, 1.0)), 0.0)
    b = batch_ref[...]
    onehot = (b == lax.broadcasted_iota(jnp.int32, (b.shape[0], g), 1)).astype(
        jnp.float32
    )
    lam2 = onehot @ lmi_ref[...]
    dinv_ref[...] = dinv
    diag_ref[...] = lam2 - 1.0
    c_ref[...] = lam2 * dinv
    z = jnp.dot(x_ref[...], wall_ref[...], preferred_element_type=jnp.float32)
    acc_ref[...] = z[:, :fo]
    y = z[:, fo:]
    y_ref[...] = y
    hs_ref[...] = (lam2 * dinv) * y


def _tc_prep(deg2, batch_pad, lmi, x, wall, fo, np_):
    g = lmi.shape[0]
    d = x.shape[1]
    fw = wall.shape[1] - fo
    nb = np_ // _BLK
    return pl.pallas_call(
        functools.partial(_prep_body, g, fo),
        grid=(nb,),
        in_specs=[
            pl.BlockSpec((NC, _BLK, 1), lambda i: (0, i, 0)),
            pl.BlockSpec((_BLK, 1), lambda i: (i, 0)),
            pl.BlockSpec((g, 1), lambda i: (0, 0)),
            pl.BlockSpec((_BLK, d), lambda i: (i, 0)),
            pl.BlockSpec((d, fo + fw), lambda i: (0, 0)),
        ],
        out_specs=[
            pl.BlockSpec((_BLK, 1), lambda i: (i, 0)),
            pl.BlockSpec((_BLK, 1), lambda i: (i, 0)),
            pl.BlockSpec((_BLK, 1), lambda i: (i, 0)),
            pl.BlockSpec((_BLK, fw), lambda i: (i, 0)),
            pl.BlockSpec((_BLK, fw), lambda i: (i, 0)),
            pl.BlockSpec((_BLK, fo), lambda i: (i, 0)),
        ],
        out_shape=[
            jax.ShapeDtypeStruct((np_, 1), jnp.float32),
            jax.ShapeDtypeStruct((np_, 1), jnp.float32),
            jax.ShapeDtypeStruct((np_, 1), jnp.float32),
            jax.ShapeDtypeStruct((np_, fw), jnp.float32),
            jax.ShapeDtypeStruct((np_, fw), jnp.float32),
            jax.ShapeDtypeStruct((np_, fo), jnp.float32),
        ],
    )(deg2, batch_pad, lmi, x, wall)


def _chain_body(alpha, beta, fo, parts_ref, tp_ref, tp2_ref, dinv_ref,
                c_ref, diag_ref, accin_ref, accout_ref, tn_ref, hs_ref,
                tp2n_ref):
    agg = parts_ref[0] + parts_ref[1]
    tp = tp_ref[...]
    lh = diag_ref[...] * tp - dinv_ref[...] * agg
    t = alpha * lh - beta * tp2_ref[...] if beta else alpha * lh
    accout_ref[...] = accin_ref[...] + t[:, :fo]
    tn = t[:, fo:]
    tn_ref[...] = tn
    hs_ref[...] = c_ref[...] * tn
    tp2n_ref[...] = tp[:, fo:]


def _tc_chain(parts, tp, tp2, dinv, cvec, diag, accin, alpha, beta, fo, np_):
    f = tp.shape[1]
    fn = f - fo
    nb = np_ // _BLK
    return pl.pallas_call(
        functools.partial(_chain_body, alpha, beta, fo),
        grid=(nb,),
        in_specs=[
            pl.BlockSpec((NC, _BLK, f), lambda i: (0, i, 0)),
            pl.BlockSpec((_BLK, f), lambda i: (i, 0)),
            pl.BlockSpec((_BLK, f), lambda i: (i, 0)),
            pl.BlockSpec((_BLK, 1), lambda i: (i, 0)),
            pl.BlockSpec((_BLK, 1), lambda i: (i, 0)),
            pl.BlockSpec((_BLK, 1), lambda i: (i, 0)),
            pl.BlockSpec((_BLK, fo), lambda i: (i, 0)),
        ],
        out_specs=[
            pl.BlockSpec((_BLK, fo), lambda i: (i, 0)),
            pl.BlockSpec((_BLK, fn), lambda i: (i, 0)),
            pl.BlockSpec((_BLK, fn), lambda i: (i, 0)),
            pl.BlockSpec((_BLK, fn), lambda i: (i, 0)),
        ],
        out_shape=[
            jax.ShapeDtypeStruct((np_, fo), jnp.float32),
            jax.ShapeDtypeStruct((np_, fn), jnp.float32),
            jax.ShapeDtypeStruct((np_, fn), jnp.float32),
            jax.ShapeDtypeStruct((np_, fn), jnp.float32),
        ],
    )(parts, tp, tp2, dinv, cvec, diag, accin)


def _step_body(alpha, beta, parts_ref, tp_ref, tp2_ref, dinv_ref,
               c_ref, diag_ref, wk_ref, accin_ref, tx_ref, hs_ref, accout_ref):
    agg = parts_ref[0] + parts_ref[1]
    tp = tp_ref[...]
    lh = diag_ref[...] * tp - dinv_ref[...] * agg
    tx = alpha * lh - beta * tp2_ref[...] if beta else alpha * lh
    tx_ref[...] = tx
    hs_ref[...] = c_ref[...] * tx
    accout_ref[...] = accin_ref[...] + jnp.dot(
        tx, wk_ref[...], preferred_element_type=jnp.float32
    )


def _tc_step(parts, tp, tp2, dinv, cvec, diag, wk, accin, alpha, beta, np_):
    f = tp.shape[1]
    fo = wk.shape[1]
    nb = np_ // _BLK
    return pl.pallas_call(
        functools.partial(_step_body, alpha, beta),
        grid=(nb,),
        in_specs=[
            pl.BlockSpec((NC, _BLK, f), lambda i: (0, i, 0)),
            pl.BlockSpec((_BLK, f), lambda i: (i, 0)),
            pl.BlockSpec((_BLK, f), lambda i: (i, 0)),
            pl.BlockSpec((_BLK, 1), lambda i: (i, 0)),
            pl.BlockSpec((_BLK, 1), lambda i: (i, 0)),
            pl.BlockSpec((_BLK, 1), lambda i: (i, 0)),
            pl.BlockSpec((f, fo), lambda i: (0, 0)),
            pl.BlockSpec((_BLK, fo), lambda i: (i, 0)),
        ],
        out_specs=[
            pl.BlockSpec((_BLK, f), lambda i: (i, 0)),
            pl.BlockSpec((_BLK, f), lambda i: (i, 0)),
            pl.BlockSpec((_BLK, fo), lambda i: (i, 0)),
        ],
        out_shape=[
            jax.ShapeDtypeStruct((np_, f), jnp.float32),
            jax.ShapeDtypeStruct((np_, f), jnp.float32),
            jax.ShapeDtypeStruct((np_, fo), jnp.float32),
        ],
    )(parts, tp, tp2, dinv, cvec, diag, wk, accin)


def _fin_body(project, parts_ref, tp_ref, tp2_ref, dinv_ref, c_ref, diag_ref,
              *rest):
    if project:
        wk_ref, accin_ref, bias_ref, wn_ref, h_ref, hs_ref, accn_ref = rest
    else:
        accin_ref, bias_ref, wn_ref, h_ref, hs_ref, accn_ref = rest
    agg = parts_ref[0] + parts_ref[1]
    lh = diag_ref[...] * tp_ref[...] - dinv_ref[...] * agg
    tx = 2.0 * lh - tp2_ref[...]
    if project:
        o = accin_ref[...] + jnp.dot(
            tx, wk_ref[...], preferred_element_type=jnp.float32
        )
    else:
        o = accin_ref[...] + tx
    h = jnp.maximum(o + bias_ref[...], 0.0)
    h_ref[...] = h
    hs_ref[...] = c_ref[...] * h
    accn_ref[...] = jnp.dot(h, wn_ref[...], preferred_element_type=jnp.float32)


def _tc_finish(parts, tp, tp2, dinv, cvec, diag, wk, accin, bias, wnext, np_):
    f = tp.shape[1]
    fo = accin.shape[1]
    fn = wnext.shape[1]
    nb = np_ // _BLK
    project = wk is not None
    in_specs = [
        pl.BlockSpec((NC, _BLK, f), lambda i: (0, i, 0)),
        pl.BlockSpec((_BLK, f), lambda i: (i, 0)),
        pl.BlockSpec((_BLK, f), lambda i: (i, 0)),
        pl.BlockSpec((_BLK, 1), lambda i: (i, 0)),
        pl.BlockSpec((_BLK, 1), lambda i: (i, 0)),
        pl.BlockSpec((_BLK, 1), lambda i: (i, 0)),
    ]
    args = [parts, tp, tp2, dinv, cvec, diag]
    if project:
        in_specs.append(pl.BlockSpec((f, fo), lambda i: (0, 0)))
        args.append(wk)
    in_specs += [
        pl.BlockSpec((_BLK, fo), lambda i: (i, 0)),
        pl.BlockSpec((1, fo), lambda i: (0, 0)),
        pl.BlockSpec((fo, fn), lambda i: (0, 0)),
    ]
    args += [accin, bias, wnext]
    return pl.pallas_call(
        functools.partial(_fin_body, project),
        grid=(nb,),
        in_specs=in_specs,
        out_specs=[
            pl.BlockSpec((_BLK, fo), lambda i: (i, 0)),
            pl.BlockSpec((_BLK, fo), lambda i: (i, 0)),
            pl.BlockSpec((_BLK, fn), lambda i: (i, 0)),
        ],
        out_shape=[
            jax.ShapeDtypeStruct((np_, fo), jnp.float32),
            jax.ShapeDtypeStruct((np_, fo), jnp.float32),
            jax.ShapeDtypeStruct((np_, fn), jnp.float32),
        ],
    )(*args)


def _pool_body(g, nb, h_ref, batch_ref, a1w_ref, a1b_ref, a2w_ref, a2b_ref,
               out_ref, acc_ref):
    i = pl.program_id(0)

    @pl.when(i == 0)
    def _init():
        acc_ref[...] = jnp.full_like(acc_ref, -jnp.inf)

    h = h_ref[...]
    b = batch_ref[...]
    for gg in range(g):
        sel = jnp.where(b == gg, h, -jnp.inf)
        acc_ref[gg, :] = jnp.maximum(acc_ref[gg, :], jnp.max(sel, axis=0))

    @pl.when(i == nb - 1)
    def _fin():
        gmax = acc_ref[...]
        gmax = jnp.where(jnp.isfinite(gmax), gmax, 0.0)
        z = jnp.maximum(
            jnp.dot(gmax, a1w_ref[...], preferred_element_type=jnp.float32)
            + a1b_ref[...],
            0.0,
        )
        out_ref[...] = (
            jnp.dot(z, a2w_ref[...], preferred_element_type=jnp.float32)
            + a2b_ref[...]
        )


def _tc_pool(h3, batch_pad, a1w, a1b, a2w, a2b, g, np_):
    f = h3.shape[1]
    nb = np_ // _BLK
    return pl.pallas_call(
        functools.partial(_pool_body, g, nb),
        grid=(nb,),
        in_specs=[
            pl.BlockSpec((_BLK, f), lambda i: (i, 0)),
            pl.BlockSpec((_BLK, 1), lambda i: (i, 0)),
            pl.BlockSpec((f, 16), lambda i: (0, 0)),
            pl.BlockSpec((1, 16), lambda i: (0, 0)),
            pl.BlockSpec((16, 1), lambda i: (0, 0)),
            pl.BlockSpec((1, 1), lambda i: (0, 0)),
        ],
        out_specs=pl.BlockSpec((g, 1), lambda i: (0, 0)),
        out_shape=jax.ShapeDtypeStruct((g, 1), jnp.float32),
        scratch_shapes=[pltpu.VMEM((g, f), jnp.float32)],
    )(h3, batch_pad, a1w, a1b, a2w, a2b)


def kernel(x, edge_index, batch, lmax, W1, b1, W2, b2, W3, b3, A1w, A1b, A2w, A2b):
    n, d = x.shape
    e = edge_index.shape[1]
    g = lmax.shape[0]
    s_order = W1.shape[0]
    fo1 = W1.shape[2]

    np_ = ((n + 16 + _BLK - 1) // _BLK) * _BLK
    unit = CHUNK * NC * NS * 8
    ecp = ((e + unit - 1) // unit) * NC * NS * 8
    pad_e = ecp * CHUNK - e

    src = edge_index[0]
    dst = edge_index[1]
    pad_idx = n + (jnp.arange(pad_e, dtype=jnp.int32) % 16)
    src2d = jnp.concatenate([src, pad_idx]).reshape(ecp, CHUNK)
    dst2d = jnp.concatenate([dst, pad_idx]).reshape(ecp, CHUNK)

    x_pad = jnp.zeros((np_, d), jnp.float32).at[:n].set(x)
    batch_pad = jnp.full((np_, 1), g, jnp.int32).at[:n, 0].set(batch)
    lmi = (2.0 / lmax).reshape(g, 1)

    widths = {16, W2.shape[1], W3.shape[1]}
    widths.update((s_order - 1 - j) * fo1 for j in range(s_order - 1))
    zeros = {f: jnp.zeros((np_, f), jnp.float32) for f in widths}

    ones16 = jnp.ones((np_, 16), jnp.float32)
    deg16 = _sc_propagate(ones16, src2d, src2d, zeros[16], np_)
    deg2 = deg16[:, :, :1]

    wall = jnp.concatenate([W1[k] for k in range(s_order)], axis=1)
    dinv, cvec, diag, tp, hs, acc = _tc_prep(
        deg2, batch_pad, lmi, x_pad, wall, fo1, np_
    )
    tp2 = tp
    for j in range(1, s_order):
        fw = (s_order - j) * fo1
        parts = _sc_propagate(hs, src2d, dst2d, zeros[fw], np_)
        alpha, beta = (1.0, 0.0) if j == 1 else (2.0, 1.0)
        if j < s_order - 1:
            acc, tn, hs, tp2n = _tc_chain(
                parts, tp, tp2, dinv, cvec, diag, acc, alpha, beta, fo1, np_
            )
            tp, tp2 = tn, tp2n
        else:
            h, hs, acc = _tc_finish(
                parts, tp, tp2, dinv, cvec, diag, None, acc,
                b1.reshape(1, -1), W2[0], np_,
            )

    for W, b, wnext in ((W2, b2, W3[0]), (W3, b3, None)):
        f_in = W.shape[1]
        tp2 = h
        tp = None
        for k in range(1, s_order):
            parts = _sc_propagate(hs, src2d, dst2d, zeros[f_in], np_)
            if k == 1:
                tp, hs, acc = _tc_step(
                    parts, tp2, tp2, dinv, cvec, diag, W[1], acc, 1.0, 0.0, np_
                )
            elif k < s_order - 1:
                tx, hs, acc = _tc_step(
                    parts, tp, tp2, dinv, cvec, diag, W[k], acc, 2.0, 1.0, np_
                )
                tp2, tp = tp, tx
            else:
                wn = wnext if wnext is not None else jnp.zeros(
                    (W.shape[2], 8), jnp.float32
                )
                h, hs, acc = _tc_finish(
                    parts, tp, tp2, dinv, cvec, diag, W[k], acc,
                    b.reshape(1, -1), wn, np_,
                )

    return _tc_pool(h, batch_pad, A1w, A1b.reshape(1, -1), A2w,
                    A2b.reshape(1, -1), g, np_)


# speedup vs baseline: 1.5305x; 1.0980x over previous
import functools

import jax
import jax.numpy as jnp
from jax import lax
from jax.experimental import pallas as pl
from jax.experimental.pallas import tpu as pltpu
from jax.experimental.pallas import tpu_sc as plsc

NC = 2
NS = 16
CHUNK = 128


@functools.cache
def _mesh():
    return plsc.VectorSubcoreMesh(
        core_axis_name="c", subcore_axis_name="s", num_cores=NC, num_subcores=NS
    )


_SB = 16


def _prop_body(cpt, np_, f, nbuf, hs_hbm, src_hbm, dst_hbm, zeros_hbm, out_hbm,
               *scratch):
    srcv, dstv = scratch[0], scratch[1]
    rows = list(scratch[2:2 + nbuf])
    acc = scratch[2 + nbuf]
    gsem = list(scratch[3 + nbuf:3 + 2 * nbuf])
    ssem = list(scratch[3 + 2 * nbuf:3 + 3 * nbuf])
    c = lax.axis_index("c")
    s = lax.axis_index("s")
    wid = c * NS + s
    nr = np_ // NS
    pltpu.sync_copy(zeros_hbm.at[pl.ds(s * nr, nr)], acc.at[pl.ds(s * nr, nr)])
    pltpu.sync_copy(src_hbm.at[pl.ds(wid * cpt, cpt)], srcv)
    pltpu.sync_copy(dst_hbm.at[pl.ds(wid * cpt, _SB)], dstv)
    plsc.subcore_barrier()

    for b in range(nbuf):
        pltpu.async_copy(hs_hbm.at[srcv.at[b]], rows[b], gsem[b])

    def body(i, carry):
        j = nbuf * i

        @pl.when(j % _SB == 0)
        def _reload():
            pltpu.sync_copy(dst_hbm.at[pl.ds(wid * cpt + j, _SB)], dstv)

        for b in range(nbuf):
            pltpu.make_async_copy(hs_hbm.at[srcv.at[j + b]], rows[b],
                                  gsem[b]).wait()
            pltpu.async_copy(rows[b], acc.at[dstv.at[j % _SB + b]], ssem[b],
                             add=True)
        for b in range(nbuf):
            pltpu.make_async_copy(rows[b], acc.at[dstv.at[j % _SB + b]],
                                  ssem[b]).wait()

            @pl.when(j + nbuf + b < cpt)
            def _n(b=b):
                pltpu.async_copy(hs_hbm.at[srcv.at[j + nbuf + b]], rows[b],
                                 gsem[b])

        return carry

    lax.fori_loop(0, cpt // nbuf, body, 0)
    plsc.subcore_barrier()
    pltpu.sync_copy(acc.at[pl.ds(s * nr, nr)], out_hbm.at[c, pl.ds(s * nr, nr)])


def _sc_propagate(hs, src2d, dst2d, zeros_f, np_):
    f = hs.shape[1]
    cpt = src2d.shape[0] // (NC * NS)
    nbuf = 2 if f > 64 else 4
    while cpt % nbuf or _SB % nbuf:
        nbuf //= 2
    k = pl.kernel(
        functools.partial(_prop_body, cpt, np_, f, nbuf),
        out_type=jax.ShapeDtypeStruct((NC, np_, f), jnp.float32),
        mesh=_mesh(),
        scratch_types=(
            [
                pltpu.VMEM((cpt, CHUNK), jnp.int32),
                pltpu.VMEM((_SB, CHUNK), jnp.int32),
            ]
            + [pltpu.VMEM((CHUNK, f), jnp.float32)] * nbuf
            + [pltpu.VMEM_SHARED((np_, f), jnp.float32)]
            + [pltpu.SemaphoreType.DMA] * (2 * nbuf)
        ),
        compiler_params=pltpu.CompilerParams(use_tc_tiling_on_sc=False),
    )
    return k(hs, src2d, dst2d, zeros_f)


_BLK = 1024


def _prep_body(g, fo, deg2_ref, batch_ref, lmi_ref, x_ref, wall_ref,
               dinv_ref, c_ref, diag_ref, y_ref, hs_ref, acc_ref):
    deg = deg2_ref[0] + deg2_ref[1]
    dinv = jnp.where(deg > 0.0, lax.rsqrt(jnp.where(deg > 0.0, deg, 1.0)), 0.0)
    b = batch_ref[...]
    onehot = (b == lax.broadcasted_iota(jnp.int32, (b.shape[0], g), 1)).astype(
        jnp.float32
    )
    lam2 = onehot @ lmi_ref[...]
    dinv_ref[...] = dinv
    diag_ref[...] = lam2 - 1.0
    c_ref[...] = lam2 * dinv
    z = jnp.dot(x_ref[...], wall_ref[...], preferred_element_type=jnp.float32)
    acc_ref[...] = z[:, :fo]
    y = z[:, fo:]
    y_ref[...] = y
    hs_ref[...] = (lam2 * dinv) * y


def _tc_prep(deg2, batch_pad, lmi, x, wall, fo, np_):
    g = lmi.shape[0]
    d = x.shape[1]
    fw = wall.shape[1] - fo
    nb = np_ // _BLK
    return pl.pallas_call(
        functools.partial(_prep_body, g, fo),
        grid=(nb,),
        in_specs=[
            pl.BlockSpec((NC, _BLK, 1), lambda i: (0, i, 0)),
            pl.BlockSpec((_BLK, 1), lambda i: (i, 0)),
            pl.BlockSpec((g, 1), lambda i: (0, 0)),
            pl.BlockSpec((_BLK, d), lambda i: (i, 0)),
            pl.BlockSpec((d, fo + fw), lambda i: (0, 0)),
        ],
        out_specs=[
            pl.BlockSpec((_BLK, 1), lambda i: (i, 0)),
            pl.BlockSpec((_BLK, 1), lambda i: (i, 0)),
            pl.BlockSpec((_BLK, 1), lambda i: (i, 0)),
            pl.BlockSpec((_BLK, fw), lambda i: (i, 0)),
            pl.BlockSpec((_BLK, fw), lambda i: (i, 0)),
            pl.BlockSpec((_BLK, fo), lambda i: (i, 0)),
        ],
        out_shape=[
            jax.ShapeDtypeStruct((np_, 1), jnp.float32),
            jax.ShapeDtypeStruct((np_, 1), jnp.float32),
            jax.ShapeDtypeStruct((np_, 1), jnp.float32),
            jax.ShapeDtypeStruct((np_, fw), jnp.float32),
            jax.ShapeDtypeStruct((np_, fw), jnp.float32),
            jax.ShapeDtypeStruct((np_, fo), jnp.float32),
        ],
    )(deg2, batch_pad, lmi, x, wall)


def _chain_body(alpha, beta, fo, parts_ref, tp_ref, tp2_ref, dinv_ref,
                c_ref, diag_ref, accin_ref, accout_ref, tn_ref, hs_ref,
                tp2n_ref):
    agg = parts_ref[0] + parts_ref[1]
    tp = tp_ref[...]
    lh = diag_ref[...] * tp - dinv_ref[...] * agg
    t = alpha * lh - beta * tp2_ref[...] if beta else alpha * lh
    accout_ref[...] = accin_ref[...] + t[:, :fo]
    tn = t[:, fo:]
    tn_ref[...] = tn
    hs_ref[...] = c_ref[...] * tn
    tp2n_ref[...] = tp[:, fo:]


def _tc_chain(parts, tp, tp2, dinv, cvec, diag, accin, alpha, beta, fo, np_):
    f = tp.shape[1]
    fn = f - fo
    nb = np_ // _BLK
    return pl.pallas_call(
        functools.partial(_chain_body, alpha, beta, fo),
        grid=(nb,),
        in_specs=[
            pl.BlockSpec((NC, _BLK, f), lambda i: (0, i, 0)),
            pl.BlockSpec((_BLK, f), lambda i: (i, 0)),
            pl.BlockSpec((_BLK, f), lambda i: (i, 0)),
            pl.BlockSpec((_BLK, 1), lambda i: (i, 0)),
            pl.BlockSpec((_BLK, 1), lambda i: (i, 0)),
            pl.BlockSpec((_BLK, 1), lambda i: (i, 0)),
            pl.BlockSpec((_BLK, fo), lambda i: (i, 0)),
        ],
        out_specs=[
            pl.BlockSpec((_BLK, fo), lambda i: (i, 0)),
            pl.BlockSpec((_BLK, fn), lambda i: (i, 0)),
            pl.BlockSpec((_BLK, fn), lambda i: (i, 0)),
            pl.BlockSpec((_BLK, fn), lambda i: (i, 0)),
        ],
        out_shape=[
            jax.ShapeDtypeStruct((np_, fo), jnp.float32),
            jax.ShapeDtypeStruct((np_, fn), jnp.float32),
            jax.ShapeDtypeStruct((np_, fn), jnp.float32),
            jax.ShapeDtypeStruct((np_, fn), jnp.float32),
        ],
    )(parts, tp, tp2, dinv, cvec, diag, accin)


def _step_body(alpha, beta, parts_ref, tp_ref, tp2_ref, dinv_ref,
               c_ref, diag_ref, wk_ref, accin_ref, tx_ref, hs_ref, accout_ref):
    agg = parts_ref[0] + parts_ref[1]
    tp = tp_ref[...]
    lh = diag_ref[...] * tp - dinv_ref[...] * agg
    tx = alpha * lh - beta * tp2_ref[...] if beta else alpha * lh
    tx_ref[...] = tx
    hs_ref[...] = c_ref[...] * tx
    accout_ref[...] = accin_ref[...] + jnp.dot(
        tx, wk_ref[...], preferred_element_type=jnp.float32
    )


def _tc_step(parts, tp, tp2, dinv, cvec, diag, wk, accin, alpha, beta, np_):
    f = tp.shape[1]
    fo = wk.shape[1]
    nb = np_ // _BLK
    return pl.pallas_call(
        functools.partial(_step_body, alpha, beta),
        grid=(nb,),
        in_specs=[
            pl.BlockSpec((NC, _BLK, f), lambda i: (0, i, 0)),
            pl.BlockSpec((_BLK, f), lambda i: (i, 0)),
            pl.BlockSpec((_BLK, f), lambda i: (i, 0)),
            pl.BlockSpec((_BLK, 1), lambda i: (i, 0)),
            pl.BlockSpec((_BLK, 1), lambda i: (i, 0)),
            pl.BlockSpec((_BLK, 1), lambda i: (i, 0)),
            pl.BlockSpec((f, fo), lambda i: (0, 0)),
            pl.BlockSpec((_BLK, fo), lambda i: (i, 0)),
        ],
        out_specs=[
            pl.BlockSpec((_BLK, f), lambda i: (i, 0)),
            pl.BlockSpec((_BLK, f), lambda i: (i, 0)),
            pl.BlockSpec((_BLK, fo), lambda i: (i, 0)),
        ],
        out_shape=[
            jax.ShapeDtypeStruct((np_, f), jnp.float32),
            jax.ShapeDtypeStruct((np_, f), jnp.float32),
            jax.ShapeDtypeStruct((np_, fo), jnp.float32),
        ],
    )(parts, tp, tp2, dinv, cvec, diag, wk, accin)


def _fin_body(project, parts_ref, tp_ref, tp2_ref, dinv_ref, c_ref, diag_ref,
              *rest):
    if project:
        wk_ref, accin_ref, bias_ref, wn_ref, h_ref, hs_ref, accn_ref = rest
    else:
        accin_ref, bias_ref, wn_ref, h_ref, hs_ref, accn_ref = rest
    agg = parts_ref[0] + parts_ref[1]
    lh = diag_ref[...] * tp_ref[...] - dinv_ref[...] * agg
    tx = 2.0 * lh - tp2_ref[...]
    if project:
        o = accin_ref[...] + jnp.dot(
            tx, wk_ref[...], preferred_element_type=jnp.float32
        )
    else:
        o = accin_ref[...] + tx
    h = jnp.maximum(o + bias_ref[...], 0.0)
    h_ref[...] = h
    hs_ref[...] = c_ref[...] * h
    accn_ref[...] = jnp.dot(h, wn_ref[...], preferred_element_type=jnp.float32)


def _tc_finish(parts, tp, tp2, dinv, cvec, diag, wk, accin, bias, wnext, np_):
    f = tp.shape[1]
    fo = accin.shape[1]
    fn = wnext.shape[1]
    nb = np_ // _BLK
    project = wk is not None
    in_specs = [
        pl.BlockSpec((NC, _BLK, f), lambda i: (0, i, 0)),
        pl.BlockSpec((_BLK, f), lambda i: (i, 0)),
        pl.BlockSpec((_BLK, f), lambda i: (i, 0)),
        pl.BlockSpec((_BLK, 1), lambda i: (i, 0)),
        pl.BlockSpec((_BLK, 1), lambda i: (i, 0)),
        pl.BlockSpec((_BLK, 1), lambda i: (i, 0)),
    ]
    args = [parts, tp, tp2, dinv, cvec, diag]
    if project:
        in_specs.append(pl.BlockSpec((f, fo), lambda i: (0, 0)))
        args.append(wk)
    in_specs += [
        pl.BlockSpec((_BLK, fo), lambda i: (i, 0)),
        pl.BlockSpec((1, fo), lambda i: (0, 0)),
        pl.BlockSpec((fo, fn), lambda i: (0, 0)),
    ]
    args += [accin, bias, wnext]
    return pl.pallas_call(
        functools.partial(_fin_body, project),
        grid=(nb,),
        in_specs=in_specs,
        out_specs=[
            pl.BlockSpec((_BLK, fo), lambda i: (i, 0)),
            pl.BlockSpec((_BLK, fo), lambda i: (i, 0)),
            pl.BlockSpec((_BLK, fn), lambda i: (i, 0)),
        ],
        out_shape=[
            jax.ShapeDtypeStruct((np_, fo), jnp.float32),
            jax.ShapeDtypeStruct((np_, fo), jnp.float32),
            jax.ShapeDtypeStruct((np_, fn), jnp.float32),
        ],
    )(*args)


def _pool_body(g, nb, h_ref, batch_ref, a1w_ref, a1b_ref, a2w_ref, a2b_ref,
               out_ref, acc_ref):
    i = pl.program_id(0)

    @pl.when(i == 0)
    def _init():
        acc_ref[...] = jnp.full_like(acc_ref, -jnp.inf)

    h = h_ref[...]
    b = batch_ref[...]
    for gg in range(g):
        sel = jnp.where(b == gg, h, -jnp.inf)
        acc_ref[gg, :] = jnp.maximum(acc_ref[gg, :], jnp.max(sel, axis=0))

    @pl.when(i == nb - 1)
    def _fin():
        gmax = acc_ref[...]
        gmax = jnp.where(jnp.isfinite(gmax), gmax, 0.0)
        z = jnp.maximum(
            jnp.dot(gmax, a1w_ref[...], preferred_element_type=jnp.float32)
            + a1b_ref[...],
            0.0,
        )
        out_ref[...] = (
            jnp.dot(z, a2w_ref[...], preferred_element_type=jnp.float32)
            + a2b_ref[...]
        )


def _tc_pool(h3, batch_pad, a1w, a1b, a2w, a2b, g, np_):
    f = h3.shape[1]
    nb = np_ // _BLK
    return pl.pallas_call(
        functools.partial(_pool_body, g, nb),
        grid=(nb,),
        in_specs=[
            pl.BlockSpec((_BLK, f), lambda i: (i, 0)),
            pl.BlockSpec((_BLK, 1), lambda i: (i, 0)),
            pl.BlockSpec((f, 16), lambda i: (0, 0)),
            pl.BlockSpec((1, 16), lambda i: (0, 0)),
            pl.BlockSpec((16, 1), lambda i: (0, 0)),
            pl.BlockSpec((1, 1), lambda i: (0, 0)),
        ],
        out_specs=pl.BlockSpec((g, 1), lambda i: (0, 0)),
        out_shape=jax.ShapeDtypeStruct((g, 1), jnp.float32),
        scratch_shapes=[pltpu.VMEM((g, f), jnp.float32)],
    )(h3, batch_pad, a1w, a1b, a2w, a2b)


def kernel(x, edge_index, batch, lmax, W1, b1, W2, b2, W3, b3, A1w, A1b, A2w, A2b):
    n, d = x.shape
    e = edge_index.shape[1]
    g = lmax.shape[0]
    s_order = W1.shape[0]
    fo1 = W1.shape[2]

    np_ = ((n + 16 + _BLK - 1) // _BLK) * _BLK
    unit = CHUNK * NC * NS * 8
    ecp = ((e + unit - 1) // unit) * NC * NS * 8
    pad_e = ecp * CHUNK - e

    src = edge_index[0]
    dst = edge_index[1]
    pad_idx = n + (jnp.arange(pad_e, dtype=jnp.int32) % 16)
    src2d = jnp.concatenate([src, pad_idx]).reshape(ecp, CHUNK)
    dst2d = jnp.concatenate([dst, pad_idx]).reshape(ecp, CHUNK)

    x_pad = jnp.zeros((np_, d), jnp.float32).at[:n].set(x)
    batch_pad = jnp.full((np_, 1), g, jnp.int32).at[:n, 0].set(batch)
    lmi = (2.0 / lmax).reshape(g, 1)

    widths = {16, W2.shape[1], W3.shape[1]}
    widths.update((s_order - 1 - j) * fo1 for j in range(s_order - 1))
    zeros = {f: jnp.zeros((np_, f), jnp.float32) for f in widths}

    ones16 = jnp.ones((np_, 16), jnp.float32)
    deg16 = _sc_propagate(ones16, src2d, src2d, zeros[16], np_)
    deg2 = deg16[:, :, :1]

    wall = jnp.concatenate([W1[k] for k in range(s_order)], axis=1)
    dinv, cvec, diag, tp, hs, acc = _tc_prep(
        deg2, batch_pad, lmi, x_pad, wall, fo1, np_
    )
    tp2 = tp
    for j in range(1, s_order):
        fw = (s_order - j) * fo1
        parts = _sc_propagate(hs, src2d, dst2d, zeros[fw], np_)
        alpha, beta = (1.0, 0.0) if j == 1 else (2.0, 1.0)
        if j < s_order - 1:
            acc, tn, hs, tp2n = _tc_chain(
                parts, tp, tp2, dinv, cvec, diag, acc, alpha, beta, fo1, np_
            )
            tp, tp2 = tn, tp2n
        else:
            h, hs, acc = _tc_finish(
                parts, tp, tp2, dinv, cvec, diag, None, acc,
                b1.reshape(1, -1), W2[0], np_,
            )

    for W, b, wnext in ((W2, b2, W3[0]), (W3, b3, None)):
        f_in = W.shape[1]
        tp2 = h
        tp = None
        for k in range(1, s_order):
            parts = _sc_propagate(hs, src2d, dst2d, zeros[f_in], np_)
            if k == 1:
                tp, hs, acc = _tc_step(
                    parts, tp2, tp2, dinv, cvec, diag, W[1], acc, 1.0, 0.0, np_
                )
            elif k < s_order - 1:
                tx, hs, acc = _tc_step(
                    parts, tp, tp2, dinv, cvec, diag, W[k], acc, 2.0, 1.0, np_
                )
                tp2, tp = tp, tx
            else:
                wn = wnext if wnext is not None else jnp.zeros(
                    (W.shape[2], 8), jnp.float32
                )
                h, hs, acc = _tc_finish(
                    parts, tp, tp2, dinv, cvec, diag, W[k], acc,
                    b.reshape(1, -1), wn, np_,
                )

    return _tc_pool(h, batch_pad, A1w, A1b.reshape(1, -1), A2w,
                    A2b.reshape(1, -1), g, np_)
